# Initial kernel scaffold; baseline (speedup 1.0000x reference)
import jax
import jax.numpy as jnp
from jax.experimental import pallas as pl


def kernel(x, edge_index, edge_attr, params):
    raise NotImplementedError("write your pallas kernel here")


# speedup vs baseline: 1.1358x; 1.1358x over previous
import functools

import jax
import jax.numpy as jnp
from jax import lax
from jax.experimental import pallas as pl
from jax.experimental.pallas import tpu as pltpu
from jax.experimental.pallas import tpu_sc as plsc

N = 10000
E = 160000
H = 64

NC = 2
NS = 16
NW = NC * NS

C = 128
EPW = 5120
E_PAD = EPW * NW
NCH = EPW // C
N_ACC = 10112
RPT = N_ACC // NS

EB = 1280
NB_E = E // EB
NB_EP = E_PAD // EB

N_HALF = 5056
EPT = E_PAD // NS
NCH2 = EPT // C
ACC_R = 5120
RPT2 = ACC_R // NS


def _bn_mlp_small(x, w1, b1, g, be, w2, b2):
    def body(x_ref, w1_ref, b1_ref, g_ref, be_ref, w2_ref, b2_ref, o_ref):
        h = jnp.dot(x_ref[...], w1_ref[...],
                    preferred_element_type=jnp.float32) + b1_ref[...]
        mu = jnp.mean(h, axis=0, keepdims=True)
        hc = h - mu
        var = jnp.mean(hc * hc, axis=0, keepdims=True)
        hn = hc * lax.rsqrt(var + 1e-5) * g_ref[...] + be_ref[...]
        hn = jnp.maximum(hn, 0.0)
        o_ref[...] = jnp.dot(hn, w2_ref[...],
                             preferred_element_type=jnp.float32) + b2_ref[...]

    r = x.shape[0]
    dout = w2.shape[1]
    return pl.pallas_call(
        body,
        out_shape=jax.ShapeDtypeStruct((r, dout), jnp.float32),
    )(x, w1, b1.reshape(1, -1), g.reshape(1, -1), be.reshape(1, -1),
      w2, b2.reshape(1, -1))


def _bn_mlp_big(x, w1, b1, g, be, w2, b2, nb_out, nb_stat, rows):
    din = x.shape[1]
    dh = w1.shape[1]
    dout = w2.shape[1]

    def stats_body(x_ref, w1_ref, b1_ref, o_ref):
        j = pl.program_id(0)
        h = jnp.dot(x_ref[...], w1_ref[...],
                    preferred_element_type=jnp.float32) + b1_ref[...]

        @pl.when(j == 0)
        def _():
            o_ref[...] = jnp.zeros_like(o_ref)

        o_ref[0:1, :] += jnp.sum(h, axis=0, keepdims=True)
        o_ref[1:2, :] += jnp.sum(h * h, axis=0, keepdims=True)

    full = lambda s: pl.BlockSpec(s, lambda j: (0,) * len(s))
    stats = pl.pallas_call(
        stats_body,
        grid=(nb_stat,),
        in_specs=[
            pl.BlockSpec((EB, din), lambda j: (j, 0)),
            full((din, dh)), full((1, dh)),
        ],
        out_specs=pl.BlockSpec((2, dh), lambda j: (0, 0)),
        out_shape=jax.ShapeDtypeStruct((2, dh), jnp.float32),
    )(x, w1, b1.reshape(1, -1))

    def apply_body(x_ref, st_ref, w1_ref, b1_ref, g_ref, be_ref, w2_ref,
                   b2_ref, o_ref):
        h = jnp.dot(x_ref[...], w1_ref[...],
                    preferred_element_type=jnp.float32) + b1_ref[...]
        mu = st_ref[0:1, :] * (1.0 / rows)
        var = st_ref[1:2, :] * (1.0 / rows) - mu * mu
        hn = (h - mu) * lax.rsqrt(var + 1e-5) * g_ref[...] + be_ref[...]
        hn = jnp.maximum(hn, 0.0)
        o_ref[...] = jnp.dot(hn, w2_ref[...],
                             preferred_element_type=jnp.float32) + b2_ref[...]

    return pl.pallas_call(
        apply_body,
        grid=(nb_out,),
        in_specs=[
            pl.BlockSpec((EB, din), lambda j: (j, 0)),
            full((2, dh)),
            full((din, dh)), full((1, dh)), full((1, dh)), full((1, dh)),
            full((dh, dout)), full((1, dout)),
        ],
        out_specs=pl.BlockSpec((EB, dout), lambda j: (j, 0)),
        out_shape=jax.ShapeDtypeStruct((nb_out * EB, dout), jnp.float32),
    )(x, stats, w1, b1.reshape(1, -1), g.reshape(1, -1), be.reshape(1, -1),
      w2, b2.reshape(1, -1))


def _mid_update(v, acc_p, lp, t_prev, first):
    def body(v_ref, ac_ref, nw1, nb1, nw2, nb2, ew1, eb1, ew2, eb2,
             tp_ref, *o_refs):
        vcur = v_ref[...][:, :H]
        agg = jnp.concatenate(
            [ac_ref[0, :N_HALF, :H], ac_ref[1, :N - N_HALF, :H]], axis=0)
        h = jnp.maximum(jnp.dot(agg, nw1[...],
                                preferred_element_type=jnp.float32) + nb1[...], 0.0)
        v_new = vcur + jnp.dot(h, nw2[...],
                               preferred_element_type=jnp.float32) + nb2[...]
        nacc = jnp.concatenate(
            [ac_ref[0, :N_HALF, H:], ac_ref[1, :N - N_HALF, H:]], axis=0)
        h2 = jnp.maximum(jnp.dot(nacc, ew1[...],
                                 preferred_element_type=jnp.float32) + eb1[...], 0.0)
        t = jnp.dot(h2, ew2[...], preferred_element_type=jnp.float32) + eb2[...]
        if first:
            o_refs[0][...] = jnp.concatenate([v_new, t], axis=1)
        else:
            o_refs[0][...] = v_new
            o_refs[1][...] = jnp.concatenate(
                [tp_ref[...] + t, jnp.zeros((N, H), jnp.float32)], axis=1)

    if first:
        out_shape = jax.ShapeDtypeStruct((N, 2 * H), jnp.float32)
    else:
        out_shape = (jax.ShapeDtypeStruct((N, H), jnp.float32),
                     jax.ShapeDtypeStruct((N, 2 * H), jnp.float32))
    np_ = lp["node"]
    ep_ = lp["edge"]
    return pl.pallas_call(
        body,
        out_shape=out_shape,
    )(v, acc_p,
      np_["w1"], np_["b1"].reshape(1, -1), np_["w2"], np_["b2"].reshape(1, -1),
      ep_["w1"], ep_["b1"].reshape(1, -1), ep_["w2"], ep_["b2"].reshape(1, -1),
      t_prev)


def _mesh():
    return plsc.VectorSubcoreMesh(core_axis_name="c", subcore_axis_name="s",
                                  num_cores=NC, num_subcores=NS)


def _zero_buf(buf, groups):
    z = jnp.zeros((16,), jnp.float32)

    def row(i, _):
        for j in range(groups):
            buf[i, pl.ds(j * 16, 16)] = z
        return 0

    lax.fori_loop(0, C, row, 0)


def _sc_gather_pass(vt, e1, src, has_t):
    @functools.partial(
        pl.kernel,
        out_type=jax.ShapeDtypeStruct((E_PAD, 2 * H), jnp.float32),
        mesh=_mesh(),
        scratch_types=[
            pltpu.VMEM((C,), jnp.int32),
            pltpu.VMEM((C, H), jnp.float32),
            pltpu.VMEM((C, 2 * H), jnp.float32),
            pltpu.VMEM((C, 2 * H), jnp.float32),
            pltpu.SemaphoreType.DMA,
        ],
    )
    def k(vt_hbm, e_hbm, src_hbm, p_hbm, sidx, ev, tv, mp, sem):
        c = lax.axis_index("c")
        s = lax.axis_index("s")
        w = c * NS + s
        ebase = w * EPW

        @pl.loop(0, NCH)
        def _(ci):
            off = ebase + ci * C
            pltpu.sync_copy(src_hbm.at[pl.ds(off, C)], sidx)
            pltpu.sync_copy(e_hbm.at[pl.ds(off, C)], ev)
            pltpu.async_copy(vt_hbm.at[sidx], tv, sem).wait()

            def row(i, _):
                for j in range(4):
                    sl = pl.ds(j * 16, 16)
                    su = pl.ds((4 + j) * 16, 16)
                    mp[i, sl] = tv[i, sl]
                    e0 = ev[i, sl]
                    if has_t:
                        e0 = e0 + tv[i, su]
                    mp[i, su] = e0
                return 0

            lax.fori_loop(0, C, row, 0)
            pltpu.sync_copy(mp, p_hbm.at[pl.ds(off, C)])

    return k(vt, e1, src)


def _sc_scatter_pass(pe, dst2):
    @functools.partial(
        pl.kernel,
        out_type=jax.ShapeDtypeStruct((NC, ACC_R, 2 * H), jnp.float32),
        mesh=_mesh(),
        scratch_types=[
            pltpu.VMEM((NCH2, C), jnp.int32),
            pltpu.VMEM((C, 2 * H), jnp.float32),
            pltpu.VMEM((C, 2 * H), jnp.float32),
            pltpu.VMEM((C, 2 * H), jnp.float32),
            pltpu.VMEM_SHARED((ACC_R, 2 * H), jnp.float32),
        ],
    )
    def k(pe_hbm, dst2_hbm, acc_hbm, dall, pv, m, zb, acc_s):
        c = lax.axis_index("c")
        s = lax.axis_index("s")

        _zero_buf(zb, 8)
        rbase = s * RPT2
        for k2 in range(2):
            pltpu.sync_copy(zb, acc_s.at[pl.ds(rbase + k2 * C, C)])
        rem = RPT2 - 2 * C
        pltpu.sync_copy(zb.at[pl.ds(0, rem)], acc_s.at[pl.ds(rbase + 2 * C, rem)])

        pltpu.sync_copy(dst2_hbm.at[pl.ds(s * NCH2, NCH2)], dall)
        base = c * N_HALF

        def tx(ci, _):
            for g in range(C // 16):
                sl = pl.ds(g * 16, 16)
                l = dall[ci, sl] - base
                ok = jnp.logical_and(l >= 0, l < N_HALF)
                dall[ci, sl] = jnp.where(ok, l, N_HALF)
            return 0

        lax.fori_loop(0, NCH2, tx, 0)
        plsc.subcore_barrier()

        ebase = s * EPT

        @pl.loop(0, NCH2)
        def _(ci):
            off = ebase + ci * C
            pltpu.sync_copy(pe_hbm.at[pl.ds(off, C)], pv)

            def row(i, _):
                for j in range(4):
                    sl = pl.ds(j * 16, 16)
                    su = pl.ds((4 + j) * 16, 16)
                    e0 = pv[i, su]
                    m[i, sl] = jnp.maximum(pv[i, sl] + e0, 0.0) + 1e-7
                    m[i, su] = jnp.maximum(e0, 0.0) + 1e-7
                return 0

            lax.fori_loop(0, C, row, 0)
            pltpu.sync_copy(m, acc_s.at[dall.at[ci]], add=True)

        plsc.subcore_barrier()

        for k2 in range(2):
            sl = pl.ds(rbase + k2 * C, C)
            pltpu.sync_copy(acc_s.at[sl], zb)
            pltpu.sync_copy(zb, acc_hbm.at[c, sl])
        sl = pl.ds(rbase + 2 * C, rem)
        pltpu.sync_copy(acc_s.at[sl], zb.at[pl.ds(0, rem)])
        pltpu.sync_copy(zb.at[pl.ds(0, rem)], acc_hbm.at[c, sl])

    return k(pe, dst2)


def _sc_gather_add(e1, t, src):
    @functools.partial(
        pl.kernel,
        out_type=jax.ShapeDtypeStruct((E_PAD, H), jnp.float32),
        mesh=_mesh(),
        scratch_types=[
            pltpu.VMEM((C,), jnp.int32),
            pltpu.VMEM((C, H), jnp.float32),
            pltpu.VMEM((C, 2 * H), jnp.float32),
            pltpu.SemaphoreType.DMA,
        ],
    )
    def k(e_hbm, t_hbm, src_hbm, eo_hbm, sidx, ev, tv, sem):
        c = lax.axis_index("c")
        s = lax.axis_index("s")
        w = c * NS + s
        ebase = w * EPW

        @pl.loop(0, NCH)
        def _(ci):
            off = ebase + ci * C
            pltpu.sync_copy(src_hbm.at[pl.ds(off, C)], sidx)
            pltpu.sync_copy(e_hbm.at[pl.ds(off, C)], ev)
            pltpu.async_copy(t_hbm.at[sidx], tv, sem).wait()

            def row(i, _):
                for j in range(4):
                    sl = pl.ds(j * 16, 16)
                    ev[i, sl] = ev[i, sl] + tv[i, sl]
                return 0

            lax.fori_loop(0, C, row, 0)
            pltpu.sync_copy(ev, eo_hbm.at[pl.ds(off, C)])

    return k(e1, t, src)


def kernel(x, edge_index, edge_attr, params):
    src = edge_index[0].astype(jnp.int32)
    dst = edge_index[1].astype(jnp.int32)
    pad = E_PAD - E
    src_p = jnp.concatenate([src, jnp.zeros((pad,), jnp.int32)])
    dst_p = jnp.concatenate([dst, jnp.full((pad,), N, jnp.int32)])
    ea_p = jnp.concatenate(
        [edge_attr, jnp.zeros((pad, edge_attr.shape[1]), jnp.float32)])

    pe = params["node_enc"]
    w2p = jnp.concatenate([pe["w2"], jnp.zeros((pe["w2"].shape[0], H),
                                               jnp.float32)], axis=1)
    b2p = jnp.concatenate([pe["b2"], jnp.zeros((H,), jnp.float32)])
    v1 = _bn_mlp_small(x, pe["w1"], pe["b1"], pe["gamma"], pe["beta"],
                       w2p, b2p)
    pe = params["edge_enc"]
    e1 = _bn_mlp_big(ea_p, pe["w1"], pe["b1"], pe["gamma"], pe["beta"],
                     pe["w2"], pe["b2"], nb_out=NB_EP, nb_stat=NB_E, rows=E)

    dst2 = dst_p.reshape(E_PAD // C, C)
    zeros_t = jnp.zeros((N, H), jnp.float32)
    p1 = _sc_gather_pass(v1, e1, src_p, has_t=False)
    acc1 = _sc_scatter_pass(p1, dst2)
    vt1 = _mid_update(v1, acc1, params["layers"][0], zeros_t, first=True)
    p2 = _sc_gather_pass(vt1, e1, src_p, has_t=True)
    acc2 = _sc_scatter_pass(p2, dst2)
    v3, tsum = _mid_update(vt1, acc2, params["layers"][1],
                           vt1[:, H:], first=False)
    e3 = _sc_gather_add(e1, tsum, src_p)

    pd = params["node_dec"]
    node_out = _bn_mlp_small(v3, pd["w1"], pd["b1"], pd["gamma"], pd["beta"],
                             pd["w2"], pd["b2"])
    pd = params["edge_dec"]
    edge_out = _bn_mlp_big(e3, pd["w1"], pd["b1"], pd["gamma"], pd["beta"],
                           pd["w2"], pd["b2"], nb_out=NB_E, nb_stat=NB_E,
                           rows=E)
    return (node_out, edge_out)


# speedup vs baseline: 1.1694x; 1.0296x over previous
import functools

import jax
import jax.numpy as jnp
from jax import lax
from jax.experimental import pallas as pl
from jax.experimental.pallas import tpu as pltpu
from jax.experimental.pallas import tpu_sc as plsc

N = 10000
E = 160000
H = 64

NC = 2
NS = 16
NW = NC * NS

C = 128
EPW = 5120
E_PAD = EPW * NW
NCH = EPW // C
N_ACC = 10112
RPT = N_ACC // NS

EB = 1280
NB_E = E // EB
NB_EP = E_PAD // EB

N_HALF = 5056
EPT = E_PAD // NS
NCH2 = EPT // C
ACC_R = 5120
RPT2 = ACC_R // NS


def _bn_mlp_small(x, w1, b1, g, be, w2, b2):
    def body(x_ref, w1_ref, b1_ref, g_ref, be_ref, w2_ref, b2_ref, o_ref):
        h = jnp.dot(x_ref[...], w1_ref[...],
                    preferred_element_type=jnp.float32) + b1_ref[...]
        mu = jnp.mean(h, axis=0, keepdims=True)
        hc = h - mu
        var = jnp.mean(hc * hc, axis=0, keepdims=True)
        hn = hc * lax.rsqrt(var + 1e-5) * g_ref[...] + be_ref[...]
        hn = jnp.maximum(hn, 0.0)
        o_ref[...] = jnp.dot(hn, w2_ref[...],
                             preferred_element_type=jnp.float32) + b2_ref[...]

    r = x.shape[0]
    dout = w2.shape[1]
    return pl.pallas_call(
        body,
        out_shape=jax.ShapeDtypeStruct((r, dout), jnp.float32),
    )(x, w1, b1.reshape(1, -1), g.reshape(1, -1), be.reshape(1, -1),
      w2, b2.reshape(1, -1))


def _bn_mlp_big(x, w1, b1, g, be, w2, b2, nb_out, nb_stat, rows):
    din = x.shape[1]
    dh = w1.shape[1]
    dout = w2.shape[1]

    def stats_body(x_ref, w1_ref, b1_ref, o_ref):
        j = pl.program_id(0)
        h = jnp.dot(x_ref[...], w1_ref[...],
                    preferred_element_type=jnp.float32) + b1_ref[...]

        @pl.when(j == 0)
        def _():
            o_ref[...] = jnp.zeros_like(o_ref)

        o_ref[0:1, :] += jnp.sum(h, axis=0, keepdims=True)
        o_ref[1:2, :] += jnp.sum(h * h, axis=0, keepdims=True)

    full = lambda s: pl.BlockSpec(s, lambda j: (0,) * len(s))
    stats = pl.pallas_call(
        stats_body,
        grid=(nb_stat,),
        in_specs=[
            pl.BlockSpec((EB, din), lambda j: (j, 0)),
            full((din, dh)), full((1, dh)),
        ],
        out_specs=pl.BlockSpec((2, dh), lambda j: (0, 0)),
        out_shape=jax.ShapeDtypeStruct((2, dh), jnp.float32),
    )(x, w1, b1.reshape(1, -1))

    def apply_body(x_ref, st_ref, w1_ref, b1_ref, g_ref, be_ref, w2_ref,
                   b2_ref, o_ref):
        h = jnp.dot(x_ref[...], w1_ref[...],
                    preferred_element_type=jnp.float32) + b1_ref[...]
        mu = st_ref[0:1, :] * (1.0 / rows)
        var = st_ref[1:2, :] * (1.0 / rows) - mu * mu
        hn = (h - mu) * lax.rsqrt(var + 1e-5) * g_ref[...] + be_ref[...]
        hn = jnp.maximum(hn, 0.0)
        o_ref[...] = jnp.dot(hn, w2_ref[...],
                             preferred_element_type=jnp.float32) + b2_ref[...]

    return pl.pallas_call(
        apply_body,
        grid=(nb_out,),
        in_specs=[
            pl.BlockSpec((EB, din), lambda j: (j, 0)),
            full((2, dh)),
            full((din, dh)), full((1, dh)), full((1, dh)), full((1, dh)),
            full((dh, dout)), full((1, dout)),
        ],
        out_specs=pl.BlockSpec((EB, dout), lambda j: (j, 0)),
        out_shape=jax.ShapeDtypeStruct((nb_out * EB, dout), jnp.float32),
    )(x, stats, w1, b1.reshape(1, -1), g.reshape(1, -1), be.reshape(1, -1),
      w2, b2.reshape(1, -1))


def _mid_update(v, acc_p, lp, t_prev, first):
    def body(v_ref, ac_ref, nw1, nb1, nw2, nb2, ew1, eb1, ew2, eb2,
             tp_ref, *o_refs):
        vcur = v_ref[...][:, :H]
        agg = jnp.concatenate(
            [ac_ref[0, :N_HALF, :H], ac_ref[1, :N - N_HALF, :H]], axis=0)
        h = jnp.maximum(jnp.dot(agg, nw1[...],
                                preferred_element_type=jnp.float32) + nb1[...], 0.0)
        v_new = vcur + jnp.dot(h, nw2[...],
                               preferred_element_type=jnp.float32) + nb2[...]
        nacc = jnp.concatenate(
            [ac_ref[0, :N_HALF, H:], ac_ref[1, :N - N_HALF, H:]], axis=0)
        h2 = jnp.maximum(jnp.dot(nacc, ew1[...],
                                 preferred_element_type=jnp.float32) + eb1[...], 0.0)
        t = jnp.dot(h2, ew2[...], preferred_element_type=jnp.float32) + eb2[...]
        if first:
            o_refs[0][...] = jnp.concatenate([v_new, t], axis=1)
        else:
            o_refs[0][...] = v_new
            o_refs[1][...] = jnp.concatenate(
                [tp_ref[...] + t, jnp.zeros((N, H), jnp.float32)], axis=1)

    if first:
        out_shape = jax.ShapeDtypeStruct((N, 2 * H), jnp.float32)
    else:
        out_shape = (jax.ShapeDtypeStruct((N, H), jnp.float32),
                     jax.ShapeDtypeStruct((N, 2 * H), jnp.float32))
    np_ = lp["node"]
    ep_ = lp["edge"]
    return pl.pallas_call(
        body,
        out_shape=out_shape,
    )(v, acc_p,
      np_["w1"], np_["b1"].reshape(1, -1), np_["w2"], np_["b2"].reshape(1, -1),
      ep_["w1"], ep_["b1"].reshape(1, -1), ep_["w2"], ep_["b2"].reshape(1, -1),
      t_prev)


def _mesh():
    return plsc.VectorSubcoreMesh(core_axis_name="c", subcore_axis_name="s",
                                  num_cores=NC, num_subcores=NS)


def _zero_buf(buf, groups):
    z = jnp.zeros((16,), jnp.float32)

    def row(i, _):
        for j in range(groups):
            buf[i, pl.ds(j * 16, 16)] = z
        return 0

    lax.fori_loop(0, C, row, 0)


def _sc_gather_pass(vt, e1, src, has_t):
    @functools.partial(
        pl.kernel,
        out_type=jax.ShapeDtypeStruct((E_PAD, 2 * H), jnp.float32),
        mesh=_mesh(),
        scratch_types=[
            pltpu.VMEM((NCH, C), jnp.int32),
            pltpu.VMEM((C, H), jnp.float32),
            pltpu.VMEM((C, 2 * H), jnp.float32),
            pltpu.SemaphoreType.DMA,
        ],
    )
    def k(vt_hbm, e_hbm, src2_hbm, p_hbm, sall, ev, tv, sem):
        c = lax.axis_index("c")
        s = lax.axis_index("s")
        w = c * NS + s
        ebase = w * EPW
        pltpu.sync_copy(src2_hbm.at[pl.ds(w * NCH, NCH)], sall)

        @pl.loop(0, NCH)
        def _(ci):
            off = ebase + ci * C
            pltpu.sync_copy(e_hbm.at[pl.ds(off, C)], ev)
            pltpu.async_copy(vt_hbm.at[sall.at[ci]], tv, sem).wait()

            def row(i, _):
                for j in range(4):
                    sl = pl.ds(j * 16, 16)
                    su = pl.ds((4 + j) * 16, 16)
                    e0 = ev[i, sl]
                    if has_t:
                        e0 = e0 + tv[i, su]
                    tv[i, sl] = jnp.maximum(tv[i, sl] + e0, 0.0) + 1e-7
                    tv[i, su] = jnp.maximum(e0, 0.0) + 1e-7
                return 0

            lax.fori_loop(0, C, row, 0)
            pltpu.sync_copy(tv, p_hbm.at[pl.ds(off, C)])

    return k(vt, e1, src)


def _sc_scatter_pass(pe, dst2):
    @functools.partial(
        pl.kernel,
        out_type=jax.ShapeDtypeStruct((NC, ACC_R, 2 * H), jnp.float32),
        mesh=_mesh(),
        scratch_types=[
            pltpu.VMEM((NCH2, C), jnp.int32),
            pltpu.VMEM((C, 2 * H), jnp.float32),
            pltpu.VMEM((C, 2 * H), jnp.float32),
            pltpu.VMEM_SHARED((ACC_R, 2 * H), jnp.float32),
        ],
    )
    def k(pe_hbm, dst2_hbm, acc_hbm, dall, pv, zb, acc_s):
        c = lax.axis_index("c")
        s = lax.axis_index("s")

        _zero_buf(zb, 8)
        rbase = s * RPT2
        for k2 in range(2):
            pltpu.sync_copy(zb, acc_s.at[pl.ds(rbase + k2 * C, C)])
        rem = RPT2 - 2 * C
        pltpu.sync_copy(zb.at[pl.ds(0, rem)], acc_s.at[pl.ds(rbase + 2 * C, rem)])

        pltpu.sync_copy(dst2_hbm.at[pl.ds(s * NCH2, NCH2)], dall)
        base = c * N_HALF

        def tx(ci, _):
            for g in range(C // 16):
                sl = pl.ds(g * 16, 16)
                l = dall[ci, sl] - base
                ok = jnp.logical_and(l >= 0, l < N_HALF)
                dall[ci, sl] = jnp.where(ok, l, N_HALF)
            return 0

        lax.fori_loop(0, NCH2, tx, 0)
        plsc.subcore_barrier()

        ebase = s * EPT

        @pl.loop(0, NCH2)
        def _(ci):
            off = ebase + ci * C
            pltpu.sync_copy(pe_hbm.at[pl.ds(off, C)], pv)
            pltpu.sync_copy(pv, acc_s.at[dall.at[ci]], add=True)

        plsc.subcore_barrier()

        for k2 in range(2):
            sl = pl.ds(rbase + k2 * C, C)
            pltpu.sync_copy(acc_s.at[sl], zb)
            pltpu.sync_copy(zb, acc_hbm.at[c, sl])
        sl = pl.ds(rbase + 2 * C, rem)
        pltpu.sync_copy(acc_s.at[sl], zb.at[pl.ds(0, rem)])
        pltpu.sync_copy(zb.at[pl.ds(0, rem)], acc_hbm.at[c, sl])

    return k(pe, dst2)


def _sc_gather_add(e1, t, src):
    @functools.partial(
        pl.kernel,
        out_type=jax.ShapeDtypeStruct((E_PAD, H), jnp.float32),
        mesh=_mesh(),
        scratch_types=[
            pltpu.VMEM((NCH, C), jnp.int32),
            pltpu.VMEM((C, H), jnp.float32),
            pltpu.VMEM((C, 2 * H), jnp.float32),
            pltpu.SemaphoreType.DMA,
        ],
    )
    def k(e_hbm, t_hbm, src_hbm, eo_hbm, sidx, ev, tv, sem):
        c = lax.axis_index("c")
        s = lax.axis_index("s")
        w = c * NS + s
        ebase = w * EPW

        pltpu.sync_copy(src_hbm.at[pl.ds(w * NCH, NCH)], sidx)

        @pl.loop(0, NCH)
        def _(ci):
            off = ebase + ci * C
            pltpu.sync_copy(e_hbm.at[pl.ds(off, C)], ev)
            pltpu.async_copy(t_hbm.at[sidx.at[ci]], tv, sem).wait()

            def row(i, _):
                for j in range(4):
                    sl = pl.ds(j * 16, 16)
                    ev[i, sl] = ev[i, sl] + tv[i, sl]
                return 0

            lax.fori_loop(0, C, row, 0)
            pltpu.sync_copy(ev, eo_hbm.at[pl.ds(off, C)])

    return k(e1, t, src)


def kernel(x, edge_index, edge_attr, params):
    src = edge_index[0].astype(jnp.int32)
    dst = edge_index[1].astype(jnp.int32)
    pad = E_PAD - E
    src_p = jnp.concatenate([src, jnp.zeros((pad,), jnp.int32)])
    dst_p = jnp.concatenate([dst, jnp.full((pad,), N, jnp.int32)])
    ea_p = jnp.concatenate(
        [edge_attr, jnp.zeros((pad, edge_attr.shape[1]), jnp.float32)])

    pe = params["node_enc"]
    w2p = jnp.concatenate([pe["w2"], jnp.zeros((pe["w2"].shape[0], H),
                                               jnp.float32)], axis=1)
    b2p = jnp.concatenate([pe["b2"], jnp.zeros((H,), jnp.float32)])
    v1 = _bn_mlp_small(x, pe["w1"], pe["b1"], pe["gamma"], pe["beta"],
                       w2p, b2p)
    pe = params["edge_enc"]
    e1 = _bn_mlp_big(ea_p, pe["w1"], pe["b1"], pe["gamma"], pe["beta"],
                     pe["w2"], pe["b2"], nb_out=NB_EP, nb_stat=NB_E, rows=E)

    dst2 = dst_p.reshape(E_PAD // C, C)
    src2 = src_p.reshape(E_PAD // C, C)
    zeros_t = jnp.zeros((N, H), jnp.float32)
    p1 = _sc_gather_pass(v1, e1, src2, has_t=False)
    acc1 = _sc_scatter_pass(p1, dst2)
    vt1 = _mid_update(v1, acc1, params["layers"][0], zeros_t, first=True)
    p2 = _sc_gather_pass(vt1, e1, src2, has_t=True)
    acc2 = _sc_scatter_pass(p2, dst2)
    v3, tsum = _mid_update(vt1, acc2, params["layers"][1],
                           vt1[:, H:], first=False)
    e3 = _sc_gather_add(e1, tsum, src2)

    pd = params["node_dec"]
    node_out = _bn_mlp_small(v3, pd["w1"], pd["b1"], pd["gamma"], pd["beta"],
                             pd["w2"], pd["b2"])
    pd = params["edge_dec"]
    edge_out = _bn_mlp_big(e3, pd["w1"], pd["b1"], pd["gamma"], pd["beta"],
                           pd["w2"], pd["b2"], nb_out=NB_E, nb_stat=NB_E,
                           rows=E)
    return (node_out, edge_out)
